# Initial kernel scaffold; baseline (speedup 1.0000x reference)
import jax
import jax.numpy as jnp
from jax.experimental import pallas as pl


def kernel(k, v, rkn_score, m_k, m_v, m_u):
    raise NotImplementedError("write your pallas kernel here")


# speedup vs baseline: 1.1527x; 1.1527x over previous
import functools

import jax
import jax.numpy as jnp
from jax import lax
from jax.experimental import pallas as pl
from jax.experimental.pallas import tpu as pltpu
from jax.experimental.pallas import tpu_sc as plsc

B, HW, M, K, C = 16, 1024, 1024, 64, 3
DECAY = 0.9
THRESHOLD = 0.05

NC, NS = 2, 16
HALF = M // 2


def _tc_body(k_ref, mk_ref, mu_row_ref, mu_col_ref,
             rank_s_ref, rank_mu_ref, pos_ref, valid_ref, msm_ref):
    kb = k_ref[0]
    mkb = mk_ref[0]
    mu_row = mu_row_ref[0]
    mu_col = mu_col_ref[0]

    logits = lax.dot_general(kb, mkb, (((1,), (1,)), ((), ())),
                             preferred_element_type=jnp.float32)
    rowmax = jnp.max(logits, axis=1, keepdims=True)
    p = jnp.exp(logits - rowmax)
    se = jnp.sum(p, axis=1, keepdims=True)
    s = p / se
    a_col = 1.0 / se
    a_row = jnp.transpose(a_col)
    msm_ref[0] = jnp.max(s, axis=0, keepdims=True)

    ii = lax.broadcasted_iota(jnp.int32, (HW, HW), 0)
    jj = lax.broadcasted_iota(jnp.int32, (HW, HW), 1)
    before = ii < jj

    take_s = (a_col < a_row) | ((a_col == a_row) & before)
    rank_s = jnp.sum(jnp.where(take_s, 1.0, 0.0), axis=0, keepdims=True)
    rank_s_ref[0] = rank_s.astype(jnp.int32)

    take_mu = (mu_col < mu_row) | ((mu_col == mu_row) & before)
    rank_mu = jnp.sum(jnp.where(take_mu, 1.0, 0.0), axis=0, keepdims=True)
    rank_mu_ref[0] = rank_mu.astype(jnp.int32)

    wv_col = a_col < THRESHOLD
    wv_row = a_row < THRESHOLD
    incl = jnp.where((ii <= jj) & wv_col, 1.0, 0.0)
    csum = jnp.sum(incl, axis=0, keepdims=True)
    countf = csum[:, HW - 1:HW]
    jrow = jj[0:1, :].astype(jnp.float32)
    posf = jnp.where(wv_row, csum - 1.0, countf + jrow - csum)
    pos_ref[0] = posf.astype(jnp.int32)
    valid_ref[0] = jnp.where(jrow < countf, 1.0, 0.0)


def _tc_stage(k, m_k, mu_row3, mu_col3):
    out_shape = [
        jax.ShapeDtypeStruct((B, 1, HW), jnp.int32),
        jax.ShapeDtypeStruct((B, 1, M), jnp.int32),
        jax.ShapeDtypeStruct((B, 1, HW), jnp.int32),
        jax.ShapeDtypeStruct((B, 1, M), jnp.float32),
        jax.ShapeDtypeStruct((B, 1, M), jnp.float32),
    ]
    return pl.pallas_call(
        _tc_body,
        grid=(B,),
        in_specs=[
            pl.BlockSpec((1, HW, K), lambda b: (b, 0, 0)),
            pl.BlockSpec((1, M, K), lambda b: (b, 0, 0)),
            pl.BlockSpec((1, 1, M), lambda b: (b, 0, 0)),
            pl.BlockSpec((1, M, 1), lambda b: (b, 0, 0)),
        ],
        out_specs=[
            pl.BlockSpec((1, 1, HW), lambda b: (b, 0, 0)),
            pl.BlockSpec((1, 1, M), lambda b: (b, 0, 0)),
            pl.BlockSpec((1, 1, HW), lambda b: (b, 0, 0)),
            pl.BlockSpec((1, 1, M), lambda b: (b, 0, 0)),
            pl.BlockSpec((1, 1, M), lambda b: (b, 0, 0)),
        ],
        out_shape=out_shape,
    )(k, m_k, mu_row3, mu_col3)


def _sc_body(catk_hbm, catv_hbm, mu_hbm, rkn_hbm, rank_s_hbm, rank_mu_hbm,
             pos_hbm, valid_hbm, msm_hbm,
             outk_hbm, outv_hbm, outu_hbm,
             rs_v, rmu_v, pos_v, val_v, msm_v, mu_v, rkn_v, catv_v,
             idx2_v, idxp_v, g_v, cidx_v, cidx2d_v, outu_v, outv_v, rows_v,
             sem):
    cid_core = lax.axis_index("c")
    sid = lax.axis_index("s")
    wid = sid * NC + cid_core
    b = wid // 2
    h = wid % 2

    pltpu.sync_copy(rank_s_hbm.at[b], rs_v)
    pltpu.sync_copy(rank_mu_hbm.at[b], rmu_v)
    pltpu.sync_copy(pos_hbm.at[b], pos_v)
    pltpu.sync_copy(valid_hbm.at[b], val_v)
    pltpu.sync_copy(msm_hbm.at[b], msm_v)
    pltpu.sync_copy(mu_hbm.at[b], mu_v)
    pltpu.sync_copy(rkn_hbm.at[b], rkn_v)
    pltpu.sync_copy(catv_hbm.at[pl.ds(b * 2 * HW, 2 * HW)], catv_v)

    iota16 = lax.iota(jnp.int32, 16)
    nch = HW // 16

    for t in range(nch):
        jv = iota16 + t * 16
        plsc.store_scatter(idx2_v, [rs_v[pl.ds(t * 16, 16)]], jv)
        plsc.store_scatter(idxp_v, [rmu_v[pl.ds(t * 16, 16)]], jv)
    for t in range(nch):
        plsc.store_scatter(g_v, [pos_v[pl.ds(t * 16, 16)]],
                           idx2_v[pl.ds(t * 16, 16)])
    for t in range(nch):
        sl = pl.ds(t * 16, 16)
        vv = val_v[sl]
        cid = jnp.where(vv > 0.5, g_v[sl], idxp_v[sl] + M)
        cidx_v[sl] = cid
        row = t // (nch // 2)
        col = (t % (nch // 2)) * 16
        cidx2d_v[row, pl.ds(col, 16)] = cid + b * (2 * HW)

    cp = pltpu.async_copy(catk_hbm.at[cidx2d_v.at[h]], rows_v, sem)

    for t in range(nch):
        sl = pl.ds(t * 16, 16)
        vv = val_v[sl]
        rk = plsc.load_gather(rkn_v, [idx2_v[sl]])
        uu = plsc.load_gather(mu_v, [idxp_v[sl]])
        outu_v[sl] = jnp.where(vv > 0.5, 1.0 + rk,
                               DECAY * uu + msm_v[sl] + rk)
        cid = cidx_v[sl]
        lrow = iota16 + t * 16
        for cc in range(C):
            ccv = jnp.full((16,), cc, jnp.int32)
            vals = plsc.load_gather(catv_v, [cid, ccv])
            plsc.store_scatter(outv_v, [lrow, ccv], vals)

    off = b * M + h * HALF
    pltpu.sync_copy(outu_v.at[pl.ds(h * HALF, HALF)],
                    outu_hbm.at[pl.ds(off, HALF)])
    pltpu.sync_copy(outv_v.at[pl.ds(h * HALF, HALF)],
                    outv_hbm.at[pl.ds(off, HALF)])
    cp.wait()
    pltpu.sync_copy(rows_v, outk_hbm.at[pl.ds(off, HALF)])


def _sc_stage(catk, catv, m_u, rkn, rank_s, rank_mu, pos, validv, msm):
    mesh = plsc.VectorSubcoreMesh(core_axis_name="c", subcore_axis_name="s")
    fn = functools.partial(
        pl.kernel,
        mesh=mesh,
        compiler_params=pltpu.CompilerParams(
            needs_layout_passes=False, use_tc_tiling_on_sc=False),
        out_type=[
            jax.ShapeDtypeStruct((B * M, K), jnp.float32),
            jax.ShapeDtypeStruct((B * M, C), jnp.float32),
            jax.ShapeDtypeStruct((B * M,), jnp.float32),
        ],
        scratch_types=[
            pltpu.VMEM((HW,), jnp.int32),
            pltpu.VMEM((M,), jnp.int32),
            pltpu.VMEM((HW,), jnp.int32),
            pltpu.VMEM((M,), jnp.float32),
            pltpu.VMEM((M,), jnp.float32),
            pltpu.VMEM((M,), jnp.float32),
            pltpu.VMEM((HW,), jnp.float32),
            pltpu.VMEM((2 * HW, C), jnp.float32),
            pltpu.VMEM((HW,), jnp.int32),
            pltpu.VMEM((M,), jnp.int32),
            pltpu.VMEM((HW,), jnp.int32),
            pltpu.VMEM((M,), jnp.int32),
            pltpu.VMEM((2, HALF), jnp.int32),
            pltpu.VMEM((M,), jnp.float32),
            pltpu.VMEM((M, C), jnp.float32),
            pltpu.VMEM((HALF, K), jnp.float32),
            pltpu.SemaphoreType.DMA,
        ],
    )(_sc_body)
    return fn(catk, catv, m_u, rkn, rank_s, rank_mu, pos, validv, msm)


def kernel(k, v, rkn_score, m_k, m_v, m_u):
    mu_row3 = m_u.reshape(B, 1, M)
    mu_col3 = m_u.reshape(B, M, 1)
    rank_s, rank_mu, pos, validv, msm = _tc_stage(k, m_k, mu_row3, mu_col3)

    catk = jnp.concatenate([k, m_k], axis=1).reshape(B * 2 * HW, K)
    catv = jnp.concatenate([v, m_v], axis=1).reshape(B * 2 * HW, C)
    rkn = rkn_score[..., 0]

    outk, outv, outu = _sc_stage(
        catk, catv, m_u, rkn,
        rank_s.reshape(B, HW), rank_mu.reshape(B, M), pos.reshape(B, HW),
        validv.reshape(B, M), msm.reshape(B, M))
    return (outk.reshape(B, M, K), outv.reshape(B, M, C), outu.reshape(B, M))
